# baseline (speedup 1.0000x reference)
import jax
import jax.numpy as jnp
from jax import lax
from jax.experimental import pallas as pl
from jax.experimental.pallas import tpu as pltpu
from jax.experimental.pallas import tpu_sc as plsc

_NUM_CORES = 2
_NUM_SUBCORES = 16
_LANES = 16

_LOG1P_C = (
    -0.05437093355557315, 0.2164487077843509, -0.46502043744559324,
    0.9959657831345089, 7.942077648755808e-05,
)


def _oddeven_merge(lo, hi, r, out):
    step = r * 2
    if step < hi - lo:
        _oddeven_merge(lo, hi, step, out)
        _oddeven_merge(lo + r, hi, step, out)
        for i in range(lo + r, hi - r, step):
            out.append((i, i + r))
    else:
        out.append((lo, lo + r))


def _oddeven_merge_sort_range(lo, hi, out):
    if (hi - lo) >= 1:
        mid = lo + ((hi - lo) // 2)
        _oddeven_merge_sort_range(lo, mid, out)
        _oddeven_merge_sort_range(mid + 1, hi, out)
        _oddeven_merge(lo, hi, 1, out)


def _bitonic_merge_list(slots, ops):
    n = len(slots)
    d = n // 2
    while d >= 1:
        for i in range(n):
            if (i & d) == 0 and i + d < n:
                ops.append(("ce", slots[i], slots[i + d]))
        d //= 2


def _build_network():
    ops = []
    pairs16 = []
    _oddeven_merge_sort_range(0, 15, pairs16)
    groups = [list(range(g, g + 16)) for g in (0, 16, 32, 48)]
    for g in groups:
        for (i, j) in pairs16:
            ops.append(("ce", g[i], g[j]))
    a, b, c, d = groups

    def merge32(x, y):
        for i in range(16):
            ops.append(("ce", x[i], y[15 - i]))
        u, l = x[:], list(reversed(y))
        _bitonic_merge_list(u, ops)
        _bitonic_merge_list(l, ops)
        return u + l

    s_ab = merge32(a, b)
    s_cd = merge32(c, d)
    for i in range(32):
        ops.append(("max", s_ab[i], s_cd[31 - i]))
    t = s_ab
    for i in range(16):
        ops.append(("ce", t[i], t[i + 16]))
    x, y = t[:16], t[16:]
    _bitonic_merge_list(x, ops)
    _bitonic_merge_list(y, ops)
    outs = (x[1], x[9], y[13])
    need = set(outs)
    keep = []
    for op in reversed(ops):
        kind, i, j = op
        if (i in need) if kind == "max" else (i in need or j in need):
            keep.append(op)
            need.add(i)
            need.add(j)
    return list(reversed(keep)), outs


_NET_OPS, (_S2, _S10, _S30) = _build_network()


def _sc_loss(pred_t, gate_t):
    n_exp, n_rows = pred_t.shape
    nw = _NUM_CORES * _NUM_SUBCORES
    cols_per = n_rows // nw
    chunk = 256
    n_chunks = cols_per // chunk
    inv_n = 1.0 / float(n_rows * n_exp)
    mesh = plsc.VectorSubcoreMesh(
        core_axis_name="c", subcore_axis_name="s",
        num_cores=_NUM_CORES, num_subcores=_NUM_SUBCORES)

    def body(pred_hbm, gate_hbm, out_hbm, g_v, p_v, res_v,
             sg0, sg1, sp0, sp1):
        cid = lax.axis_index("c")
        sid = lax.axis_index("s")
        wid = sid * _NUM_CORES + cid
        base = wid * cols_per

        def g_copy(cidx, b, sem):
            return pltpu.make_async_copy(
                gate_hbm.at[:, pl.ds(base + cidx * chunk, chunk)],
                g_v.at[b], sem)

        def p_copy(cidx, b, sem):
            return pltpu.make_async_copy(
                pred_hbm.at[:, pl.ds(base + cidx * chunk, chunk)],
                p_v.at[b], sem)

        def group_body(b, t, acc):
            col = t * _LANES
            vals = [None] * n_exp

            def val(e):
                if vals[e] is None:
                    vals[e] = g_v[b, e, pl.ds(col, _LANES)]
                return vals[e]

            for kind, i, j in _NET_OPS:
                if kind == "ce":
                    hi = jnp.maximum(val(i), val(j))
                    lo = jnp.minimum(val(i), val(j))
                    vals[i], vals[j] = hi, lo
                else:
                    vals[i] = jnp.maximum(val(i), val(j))
            v2, v10, v30 = vals[_S2], vals[_S10], vals[_S30]
            for e in range(n_exp):
                g = g_v[b, e, pl.ds(col, _LANES)]
                p = p_v[b, e, pl.ds(col, _LANES)]
                coeff = (jnp.where(g >= v30, 1.5, 0.5)
                         + jnp.where(g >= v10, 1.5, 0.0))
                ex = jnp.exp(-jnp.abs(p))
                poly = jnp.full((_LANES,), _LOG1P_C[0], jnp.float32)
                for cf in _LOG1P_C[1:]:
                    poly = poly * ex + cf
                sp = jnp.maximum(p, 0.0) + poly
                acc = acc + coeff * sp
                acc = acc - 3.0 * jnp.where(g >= v2, p, 0.0)
            return acc

        def chunk_groups(b, acc):
            return lax.fori_loop(
                0, chunk // _LANES, lambda t, ac: group_body(b, t, ac), acc,
                unroll=2)

        g_copy(0, 0, sg0).start()
        p_copy(0, 0, sp0).start()
        g_copy(1, 1, sg1).start()
        p_copy(1, 1, sp1).start()

        def outer(j, acc):
            c0 = j * 2

            g_copy(c0, 0, sg0).wait()
            p_copy(c0, 0, sp0).wait()
            acc = chunk_groups(0, acc)

            @pl.when(c0 + 2 < n_chunks)
            def _():
                g_copy(c0 + 2, 0, sg0).start()
                p_copy(c0 + 2, 0, sp0).start()

            g_copy(c0 + 1, 1, sg1).wait()
            p_copy(c0 + 1, 1, sp1).wait()
            acc = chunk_groups(1, acc)

            @pl.when(c0 + 3 < n_chunks)
            def _():
                g_copy(c0 + 3, 1, sg1).start()
                p_copy(c0 + 3, 1, sp1).start()

            return acc

        acc = jnp.zeros((_LANES,), jnp.float32)
        acc = lax.fori_loop(0, n_chunks // 2, outer, acc)

        tile_total = jnp.sum(acc) * inv_n
        res_v[...] = jnp.full((_LANES,), tile_total, jnp.float32)
        pltpu.sync_copy(res_v, out_hbm.at[wid])

    out = pl.kernel(
        body,
        out_type=jax.ShapeDtypeStruct((nw, _LANES), jnp.float32),
        mesh=mesh,
        scratch_types=[
            pltpu.VMEM((2, n_exp, chunk), jnp.float32),
            pltpu.VMEM((2, n_exp, chunk), jnp.float32),
            pltpu.VMEM((_LANES,), jnp.float32),
            pltpu.SemaphoreType.DMA,
            pltpu.SemaphoreType.DMA,
            pltpu.SemaphoreType.DMA,
            pltpu.SemaphoreType.DMA,
        ],
        compiler_params=pltpu.CompilerParams(needs_layout_passes=False),
    )(pred_t, gate_t)
    return out


def kernel(predictions, gate_logits):
    out = _sc_loss(predictions.T, gate_logits.T)
    return jnp.sum(out[:, 0])


# speedup vs baseline: 2.0159x; 2.0159x over previous
import jax
import jax.numpy as jnp
from jax import lax
from jax.experimental import pallas as pl
from jax.experimental.pallas import tpu as pltpu
from jax.experimental.pallas import tpu_sc as plsc

_NUM_CORES = 2
_NUM_SUBCORES = 16
_LANES = 16

_LOG1P_C = (
    -0.05437093355557315, 0.2164487077843509, -0.46502043744559324,
    0.9959657831345089, 7.942077648755808e-05,
)


def _oddeven_merge(lo, hi, r, out):
    step = r * 2
    if step < hi - lo:
        _oddeven_merge(lo, hi, step, out)
        _oddeven_merge(lo + r, hi, step, out)
        for i in range(lo + r, hi - r, step):
            out.append((i, i + r))
    else:
        out.append((lo, lo + r))


def _oddeven_merge_sort_range(lo, hi, out):
    if (hi - lo) >= 1:
        mid = lo + ((hi - lo) // 2)
        _oddeven_merge_sort_range(lo, mid, out)
        _oddeven_merge_sort_range(mid + 1, hi, out)
        _oddeven_merge(lo, hi, 1, out)


def _bitonic_merge_list(slots, ops):
    n = len(slots)
    d = n // 2
    while d >= 1:
        for i in range(n):
            if (i & d) == 0 and i + d < n:
                ops.append(("ce", slots[i], slots[i + d]))
        d //= 2


def _build_network():
    ops = []
    pairs16 = []
    _oddeven_merge_sort_range(0, 15, pairs16)
    groups = [list(range(g, g + 16)) for g in (0, 16, 32, 48)]

    def sort16(g):
        for (i, j) in pairs16:
            ops.append(("ce", g[i], g[j]))

    a, b, c, d = groups

    def merge32(x, y):
        for i in range(16):
            ops.append(("ce", x[i], y[15 - i]))
        u, l = x[:], list(reversed(y))
        _bitonic_merge_list(u, ops)
        _bitonic_merge_list(l, ops)
        return u + l

    sort16(a)
    sort16(b)
    s_ab = merge32(a, b)
    sort16(c)
    sort16(d)
    s_cd = merge32(c, d)
    for i in range(32):
        ops.append(("max", s_ab[i], s_cd[31 - i]))
    t = s_ab
    for i in range(16):
        ops.append(("ce", t[i], t[i + 16]))
    x, y = t[:16], t[16:]
    _bitonic_merge_list(x, ops)
    _bitonic_merge_list(y, ops)
    outs = (x[1], x[9], y[13])
    need = set(outs)
    keep = []
    for op in reversed(ops):
        kind, i, j = op
        if (i in need) if kind == "max" else (i in need or j in need):
            keep.append(op)
            need.add(i)
            need.add(j)
    return list(reversed(keep)), outs


_NET_OPS, (_S2, _S10, _S30) = _build_network()


def _sc_loss(pred_t, gate_t):
    n_exp, n_rows = pred_t.shape
    nw = _NUM_CORES * _NUM_SUBCORES
    cols_per = n_rows // nw
    chunk = 256
    n_chunks = cols_per // chunk
    inv_n = 1.0 / float(n_rows * n_exp)
    mesh = plsc.VectorSubcoreMesh(
        core_axis_name="c", subcore_axis_name="s",
        num_cores=_NUM_CORES, num_subcores=_NUM_SUBCORES)

    def body(pred_hbm, gate_hbm, out_hbm, g_v, p_v, res_v,
             sg0, sg1, sp0, sp1):
        cid = lax.axis_index("c")
        sid = lax.axis_index("s")
        wid = sid * _NUM_CORES + cid
        base = wid * cols_per

        def g_copy(cidx, b, sem):
            return pltpu.make_async_copy(
                gate_hbm.at[:, pl.ds(base + cidx * chunk, chunk)],
                g_v.at[b], sem)

        def p_copy(cidx, b, sem):
            return pltpu.make_async_copy(
                pred_hbm.at[:, pl.ds(base + cidx * chunk, chunk)],
                p_v.at[b], sem)

        def group_body(b, t, acc):
            col = t * _LANES
            vals = [None] * n_exp

            def val(e):
                if vals[e] is None:
                    vals[e] = g_v[b, e, pl.ds(col, _LANES)]
                return vals[e]

            for kind, i, j in _NET_OPS:
                if kind == "ce":
                    hi = jnp.maximum(val(i), val(j))
                    lo = jnp.minimum(val(i), val(j))
                    vals[i], vals[j] = hi, lo
                else:
                    vals[i] = jnp.maximum(val(i), val(j))
            v2, v10, v30 = vals[_S2], vals[_S10], vals[_S30]
            for e in range(n_exp):
                g = g_v[b, e, pl.ds(col, _LANES)]
                p = p_v[b, e, pl.ds(col, _LANES)]
                coeff = (jnp.where(g >= v30, 1.5, 0.5)
                         + jnp.where(g >= v10, 1.5, 0.0))
                ex = jnp.exp(-jnp.abs(p))
                poly = jnp.full((_LANES,), _LOG1P_C[0], jnp.float32)
                for cf in _LOG1P_C[1:]:
                    poly = poly * ex + cf
                sp = jnp.maximum(p, 0.0) + poly
                acc = acc + coeff * sp
                acc = acc - 3.0 * jnp.where(g >= v2, p, 0.0)
            return acc

        def chunk_groups(b, acc):
            return lax.fori_loop(
                0, chunk // _LANES, lambda t, ac: group_body(b, t, ac), acc)

        g_copy(0, 0, sg0).start()
        p_copy(0, 0, sp0).start()
        g_copy(1, 1, sg1).start()
        p_copy(1, 1, sp1).start()

        def outer(j, acc):
            c0 = j * 2

            g_copy(c0, 0, sg0).wait()
            p_copy(c0, 0, sp0).wait()
            acc = chunk_groups(0, acc)

            @pl.when(c0 + 2 < n_chunks)
            def _():
                g_copy(c0 + 2, 0, sg0).start()
                p_copy(c0 + 2, 0, sp0).start()

            g_copy(c0 + 1, 1, sg1).wait()
            p_copy(c0 + 1, 1, sp1).wait()
            acc = chunk_groups(1, acc)

            @pl.when(c0 + 3 < n_chunks)
            def _():
                g_copy(c0 + 3, 1, sg1).start()
                p_copy(c0 + 3, 1, sp1).start()

            return acc

        acc = jnp.zeros((_LANES,), jnp.float32)
        acc = lax.fori_loop(0, n_chunks // 2, outer, acc)

        tile_total = jnp.sum(acc) * inv_n
        res_v[...] = jnp.full((_LANES,), tile_total, jnp.float32)
        pltpu.sync_copy(res_v, out_hbm.at[wid])

    out = pl.kernel(
        body,
        out_type=jax.ShapeDtypeStruct((nw, _LANES), jnp.float32),
        mesh=mesh,
        scratch_types=[
            pltpu.VMEM((2, n_exp, chunk), jnp.float32),
            pltpu.VMEM((2, n_exp, chunk), jnp.float32),
            pltpu.VMEM((_LANES,), jnp.float32),
            pltpu.SemaphoreType.DMA,
            pltpu.SemaphoreType.DMA,
            pltpu.SemaphoreType.DMA,
            pltpu.SemaphoreType.DMA,
        ],
        compiler_params=pltpu.CompilerParams(needs_layout_passes=False),
    )(pred_t, gate_t)
    return out


def kernel(predictions, gate_logits):
    out = _sc_loss(predictions.T, gate_logits.T)
    return jnp.sum(out[:, 0])


# speedup vs baseline: 2.0865x; 1.0350x over previous
import jax
import jax.numpy as jnp
from jax import lax
from jax.experimental import pallas as pl
from jax.experimental.pallas import tpu as pltpu
from jax.experimental.pallas import tpu_sc as plsc

_NUM_CORES = 2
_NUM_SUBCORES = 16
_LANES = 16

_LOG1P_C = (
    -0.05437093355557315, 0.2164487077843509, -0.46502043744559324,
    0.9959657831345089, 7.942077648755808e-05,
)


def _oddeven_merge(lo, hi, r, out):
    step = r * 2
    if step < hi - lo:
        _oddeven_merge(lo, hi, step, out)
        _oddeven_merge(lo + r, hi, step, out)
        for i in range(lo + r, hi - r, step):
            out.append((i, i + r))
    else:
        out.append((lo, lo + r))


def _oddeven_merge_sort_range(lo, hi, out):
    if (hi - lo) >= 1:
        mid = lo + ((hi - lo) // 2)
        _oddeven_merge_sort_range(lo, mid, out)
        _oddeven_merge_sort_range(mid + 1, hi, out)
        _oddeven_merge(lo, hi, 1, out)


def _bitonic_merge_list(slots, ops):
    n = len(slots)
    d = n // 2
    while d >= 1:
        for i in range(n):
            if (i & d) == 0 and i + d < n:
                ops.append(("ce", slots[i], slots[i + d]))
        d //= 2


def _build_network():
    ops = []
    pairs16 = []
    _oddeven_merge_sort_range(0, 15, pairs16)
    groups = [list(range(g, g + 16)) for g in (0, 16, 32, 48)]
    for g in groups:
        for (i, j) in pairs16:
            ops.append(("ce", g[i], g[j]))
    a, b, c, d = groups

    def merge32(x, y):
        for i in range(16):
            ops.append(("ce", x[i], y[15 - i]))
        u, l = x[:], list(reversed(y))
        _bitonic_merge_list(u, ops)
        _bitonic_merge_list(l, ops)
        return u + l

    s_ab = merge32(a, b)
    s_cd = merge32(c, d)
    for i in range(32):
        ops.append(("max", s_ab[i], s_cd[31 - i]))
    t = s_ab
    for i in range(16):
        ops.append(("ce", t[i], t[i + 16]))
    x, y = t[:16], t[16:]
    _bitonic_merge_list(x, ops)
    _bitonic_merge_list(y, ops)
    outs = (x[1], x[9], y[13])
    need = set(outs)
    keep = []
    for op in reversed(ops):
        kind, i, j = op
        if (i in need) if kind == "max" else (i in need or j in need):
            keep.append(op)
            need.add(i)
            need.add(j)
    return list(reversed(keep)), outs


_NET_OPS, (_S2, _S10, _S30) = _build_network()


def _sc_loss(pred_t, gate_t):
    n_exp, n_rows = pred_t.shape
    nw = _NUM_CORES * _NUM_SUBCORES
    cols_per = n_rows // nw
    chunk = 256
    n_chunks = cols_per // chunk
    inv_n = 1.0 / float(n_rows * n_exp)
    mesh = plsc.VectorSubcoreMesh(
        core_axis_name="c", subcore_axis_name="s",
        num_cores=_NUM_CORES, num_subcores=_NUM_SUBCORES)

    def body(pred_hbm, gate_hbm, out_hbm, g_v, p_v, res_v,
             sg0, sg1, sp0, sp1):
        cid = lax.axis_index("c")
        sid = lax.axis_index("s")
        wid = sid * _NUM_CORES + cid
        base = wid * cols_per

        def g_copy(cidx, b, sem):
            return pltpu.make_async_copy(
                gate_hbm.at[:, pl.ds(base + cidx * chunk, chunk)],
                g_v.at[b], sem)

        def p_copy(cidx, b, sem):
            return pltpu.make_async_copy(
                pred_hbm.at[:, pl.ds(base + cidx * chunk, chunk)],
                p_v.at[b], sem)

        def group_body(b, t, acc):
            col = t * _LANES
            vals = [None] * n_exp

            def val(e):
                if vals[e] is None:
                    vals[e] = g_v[b, e, pl.ds(col, _LANES)]
                return vals[e]

            for kind, i, j in _NET_OPS:
                if kind == "ce":
                    hi = jnp.maximum(val(i), val(j))
                    lo = jnp.minimum(val(i), val(j))
                    vals[i], vals[j] = hi, lo
                else:
                    vals[i] = jnp.maximum(val(i), val(j))
            v2, v10, v30 = vals[_S2], vals[_S10], vals[_S30]
            for e in range(n_exp):
                g = g_v[b, e, pl.ds(col, _LANES)]
                p = p_v[b, e, pl.ds(col, _LANES)]
                coeff = (jnp.where(g >= v30, 1.5, 0.5)
                         + jnp.where(g >= v10, 1.5, 0.0))
                ex = jnp.exp(-jnp.abs(p))
                poly = jnp.full((_LANES,), _LOG1P_C[0], jnp.float32)
                for cf in _LOG1P_C[1:]:
                    poly = poly * ex + cf
                sp = jnp.maximum(p, 0.0) + poly
                acc = acc + coeff * sp
                acc = acc - 3.0 * jnp.where(g >= v2, p, 0.0)
            return acc

        def chunk_groups(b, acc):
            return lax.fori_loop(
                0, chunk // _LANES, lambda t, ac: group_body(b, t, ac), acc)

        g_copy(0, 0, sg0).start()
        p_copy(0, 0, sp0).start()
        g_copy(1, 1, sg1).start()
        p_copy(1, 1, sp1).start()

        def outer(j, acc):
            c0 = j * 2

            g_copy(c0, 0, sg0).wait()
            p_copy(c0, 0, sp0).wait()
            acc = chunk_groups(0, acc)

            @pl.when(c0 + 2 < n_chunks)
            def _():
                g_copy(c0 + 2, 0, sg0).start()
                p_copy(c0 + 2, 0, sp0).start()

            g_copy(c0 + 1, 1, sg1).wait()
            p_copy(c0 + 1, 1, sp1).wait()
            acc = chunk_groups(1, acc)

            @pl.when(c0 + 3 < n_chunks)
            def _():
                g_copy(c0 + 3, 1, sg1).start()
                p_copy(c0 + 3, 1, sp1).start()

            return acc

        acc = jnp.zeros((_LANES,), jnp.float32)
        acc = lax.fori_loop(0, n_chunks // 2, outer, acc)

        tile_total = jnp.sum(acc) * inv_n
        res_v[...] = jnp.full((_LANES,), tile_total, jnp.float32)
        pltpu.sync_copy(res_v, out_hbm.at[wid])

    out = pl.kernel(
        body,
        out_type=jax.ShapeDtypeStruct((nw, _LANES), jnp.float32),
        mesh=mesh,
        scratch_types=[
            pltpu.VMEM((2, n_exp, chunk), jnp.float32),
            pltpu.VMEM((2, n_exp, chunk), jnp.float32),
            pltpu.VMEM((_LANES,), jnp.float32),
            pltpu.SemaphoreType.DMA,
            pltpu.SemaphoreType.DMA,
            pltpu.SemaphoreType.DMA,
            pltpu.SemaphoreType.DMA,
        ],
        compiler_params=pltpu.CompilerParams(needs_layout_passes=False),
    )(pred_t, gate_t)
    return out


def kernel(predictions, gate_logits):
    out = _sc_loss(predictions.T, gate_logits.T)
    return jnp.sum(out[:, 0])
